# baseline (speedup 1.0000x reference)
import jax
import jax.numpy as jnp
from jax.experimental import pallas as pl
from jax.experimental.pallas import tpu as pltpu


def _matmul_kernel(x_ref, a_ref, o_ref):
    o_ref[...] = jnp.dot(
        x_ref[...],
        a_ref[...],
        preferred_element_type=jnp.float32,
    ).astype(o_ref.dtype)


@jax.jit
def kernel(x, A):
    N, C, V, L = x.shape
    V2, W = A.shape
    assert V == V2
    M = N * C * L

    x2 = jnp.transpose(x, (0, 1, 3, 2)).reshape(M, V)

    tr = min(5464, M)
    grid = pl.cdiv(M, tr)

    out2 = pl.pallas_call(
        _matmul_kernel,
        out_shape=jax.ShapeDtypeStruct((M, W), jnp.float32),
        grid=(grid,),
        in_specs=[
            pl.BlockSpec((tr, V), lambda i: (i, 0)),
            pl.BlockSpec((V, W), lambda i: (0, 0)),
        ],
        out_specs=pl.BlockSpec((tr, W), lambda i: (i, 0)),
        compiler_params=pltpu.CompilerParams(
            dimension_semantics=("parallel",),
            vmem_limit_bytes=int(32 << 20),
        ),
    )(x2, A)

    return out2.reshape(N, C, L, W).transpose(0, 1, 3, 2)


# speedup vs baseline: 1.0543x; 1.0543x over previous
import jax
import jax.numpy as jnp
from jax.experimental import pallas as pl
from jax.experimental.pallas import tpu as pltpu


def _matmul_kernel(x_ref, a_ref, o_ref):
    o_ref[...] = jnp.dot(
        x_ref[...],
        a_ref[...],
        preferred_element_type=jnp.float32,
    ).astype(o_ref.dtype)


@jax.jit
def kernel(x, A):
    N, C, V, L = x.shape
    V2, W = A.shape
    assert V == V2
    M = N * C * L

    x2 = jnp.transpose(x, (0, 1, 3, 2)).reshape(M, V)

    tr = min(6144, M)
    grid = pl.cdiv(M, tr)

    out2 = pl.pallas_call(
        _matmul_kernel,
        out_shape=jax.ShapeDtypeStruct((M, W), jnp.float32),
        grid=(grid,),
        in_specs=[
            pl.BlockSpec((tr, V), lambda i: (i, 0)),
            pl.BlockSpec((V, W), lambda i: (0, 0)),
        ],
        out_specs=pl.BlockSpec((tr, W), lambda i: (i, 0)),
        compiler_params=pltpu.CompilerParams(
            dimension_semantics=("parallel",),
            vmem_limit_bytes=int(32 << 20),
        ),
    )(x2, A)

    return out2.reshape(N, C, L, W).transpose(0, 1, 3, 2)
